# baseline (speedup 1.0000x reference)
import functools
import math

import jax
import jax.numpy as jnp
from jax.experimental import pallas as pl

C = 384
NH = 8
DH = C // NH
T = 3
NT = 1024
N = T * NT
WIN = 10
HI = jax.lax.Precision.HIGHEST


def _qkv_body(f_ref, t_ref, w_ref, b_ref, o_ref):
    x = f_ref[:] + t_ref[:]
    o_ref[:] = jnp.dot(x, w_ref[:], precision=HI) + b_ref[:]


def _attn_body(q_ref, k_ref, v_ref, wq_ref, wk_ref, o_ref):
    q = q_ref[0]
    k = k_ref[0]
    v = v_ref[0]
    s = jax.lax.dot_general(q, k, (((1,), (1,)), ((), ())), precision=HI)
    s = s * (1.0 / math.sqrt(DH))
    m = wq_ref[:] == wk_ref[:]
    s = jnp.where(m, s, -1e9)
    s = s - jnp.max(s, axis=-1, keepdims=True)
    e = jnp.exp(s)
    p = e / jnp.sum(e, axis=-1, keepdims=True)
    o_ref[0] = jnp.dot(p, v, precision=HI)


def _fuse_body(a_ref, f_ref, t_ref, wp_ref, bp_ref, g_ref, be_ref,
               w1_ref, b1_ref, w2_ref, b2_ref, o_ref):
    a = jnp.dot(a_ref[:], wp_ref[:], precision=HI) + bp_ref[:]
    xres = a + f_ref[:] + t_ref[:]
    mu = jnp.mean(xres, axis=-1, keepdims=True)
    var = jnp.mean((xres - mu) ** 2, axis=-1, keepdims=True)
    xn = (xres - mu) / jnp.sqrt(var + 1e-5) * g_ref[:] + be_ref[:]
    h = jnp.maximum(jnp.dot(xn, w1_ref[:], precision=HI) + b1_ref[:], 0.0)
    o_ref[:] = xn + jnp.dot(h, w2_ref[:], precision=HI) + b2_ref[:]


def kernel(feats_t0, feats_t1, feats_t2, indices_t0, indices_t1, indices_t2,
           time_emb, Wqkv, bqkv, Wproj, bproj, gamma, beta, W1, b1, W2, b2,
           current_frame_idx):
    idx = jnp.concatenate([indices_t0, indices_t1, indices_t2], axis=0)
    shift = WIN // 2
    wb = idx[:, 0]
    wz = idx[:, 1]
    wy = (idx[:, 2] + shift) // WIN
    wx = (idx[:, 3] + shift) // WIN
    wid = (((wb * 4096 + wz) * 4096 + wy) * 4096 + wx).astype(jnp.int32)

    feats = jnp.concatenate([feats_t0, feats_t1, feats_t2], axis=0)
    temb = jnp.repeat(time_emb, NT, axis=0)

    qkv = pl.pallas_call(
        _qkv_body,
        out_shape=jax.ShapeDtypeStruct((N, 3 * C), jnp.float32),
    )(feats, temb, Wqkv, bqkv.reshape(1, 3 * C))

    qkv4 = qkv.reshape(N, 3, NH, DH).transpose(1, 2, 0, 3)
    q = qkv4[0, :, 2 * NT:, :]
    k = qkv4[1]
    v = qkv4[2]
    widq = wid[2 * NT:].reshape(NT, 1)
    widk = wid.reshape(1, N)

    attn = pl.pallas_call(
        _attn_body,
        grid=(NH,),
        in_specs=[
            pl.BlockSpec((1, NT, DH), lambda h: (h, 0, 0)),
            pl.BlockSpec((1, N, DH), lambda h: (h, 0, 0)),
            pl.BlockSpec((1, N, DH), lambda h: (h, 0, 0)),
            pl.BlockSpec((NT, 1), lambda h: (0, 0)),
            pl.BlockSpec((1, N), lambda h: (0, 0)),
        ],
        out_specs=pl.BlockSpec((1, NT, DH), lambda h: (h, 0, 0)),
        out_shape=jax.ShapeDtypeStruct((NH, NT, DH), jnp.float32),
    )(q, k, v, widq, widk)

    attn2 = attn.transpose(1, 0, 2).reshape(NT, C)

    out = pl.pallas_call(
        _fuse_body,
        out_shape=jax.ShapeDtypeStruct((NT, C), jnp.float32),
    )(attn2, feats_t2, time_emb[2:3], Wproj, bproj.reshape(1, C),
      gamma.reshape(1, C), beta.reshape(1, C), W1, b1.reshape(1, 2 * C),
      W2, b2.reshape(1, C))

    return out, indices_t2


# speedup vs baseline: 2.6796x; 2.6796x over previous
import functools
import math

import jax
import jax.numpy as jnp
from jax.experimental import pallas as pl

C = 384
NH = 8
DH = C // NH
T = 3
NT = 1024
N = T * NT
WIN = 10
HI = jax.lax.Precision.DEFAULT


def _qkv_body(f_ref, t_ref, w_ref, b_ref, o_ref):
    x = f_ref[:] + t_ref[:]
    o_ref[:] = jnp.dot(x, w_ref[:], precision=HI) + b_ref[:]


def _attn_body(q_ref, k_ref, v_ref, wq_ref, wk_ref, o_ref):
    q = q_ref[0]
    k = k_ref[0]
    v = v_ref[0]
    s = jax.lax.dot_general(q, k, (((1,), (1,)), ((), ())), precision=HI)
    s = s * (1.0 / math.sqrt(DH))
    m = wq_ref[:] == wk_ref[:]
    s = jnp.where(m, s, -1e9)
    s = s - jnp.max(s, axis=-1, keepdims=True)
    e = jnp.exp(s)
    p = e / jnp.sum(e, axis=-1, keepdims=True)
    o_ref[0] = jnp.dot(p, v, precision=HI)


def _fuse_body(a_ref, f_ref, t_ref, wp_ref, bp_ref, g_ref, be_ref,
               w1_ref, b1_ref, w2_ref, b2_ref, o_ref):
    a = jnp.dot(a_ref[:], wp_ref[:], precision=HI) + bp_ref[:]
    xres = a + f_ref[:] + t_ref[:]
    mu = jnp.mean(xres, axis=-1, keepdims=True)
    var = jnp.mean((xres - mu) ** 2, axis=-1, keepdims=True)
    xn = (xres - mu) / jnp.sqrt(var + 1e-5) * g_ref[:] + be_ref[:]
    h = jnp.maximum(jnp.dot(xn, w1_ref[:], precision=HI) + b1_ref[:], 0.0)
    o_ref[:] = xn + jnp.dot(h, w2_ref[:], precision=HI) + b2_ref[:]


def kernel(feats_t0, feats_t1, feats_t2, indices_t0, indices_t1, indices_t2,
           time_emb, Wqkv, bqkv, Wproj, bproj, gamma, beta, W1, b1, W2, b2,
           current_frame_idx):
    idx = jnp.concatenate([indices_t0, indices_t1, indices_t2], axis=0)
    shift = WIN // 2
    wb = idx[:, 0]
    wz = idx[:, 1]
    wy = (idx[:, 2] + shift) // WIN
    wx = (idx[:, 3] + shift) // WIN
    wid = (((wb * 4096 + wz) * 4096 + wy) * 4096 + wx).astype(jnp.int32)

    feats = jnp.concatenate([feats_t0, feats_t1, feats_t2], axis=0)
    temb = jnp.repeat(time_emb, NT, axis=0)

    qkv = pl.pallas_call(
        _qkv_body,
        out_shape=jax.ShapeDtypeStruct((N, 3 * C), jnp.float32),
    )(feats, temb, Wqkv, bqkv.reshape(1, 3 * C))

    qkv4 = qkv.reshape(N, 3, NH, DH).transpose(1, 2, 0, 3)
    q = qkv4[0, :, 2 * NT:, :]
    k = qkv4[1]
    v = qkv4[2]
    widq = wid[2 * NT:].reshape(NT, 1)
    widk = wid.reshape(1, N)

    attn = pl.pallas_call(
        _attn_body,
        grid=(NH,),
        in_specs=[
            pl.BlockSpec((1, NT, DH), lambda h: (h, 0, 0)),
            pl.BlockSpec((1, N, DH), lambda h: (h, 0, 0)),
            pl.BlockSpec((1, N, DH), lambda h: (h, 0, 0)),
            pl.BlockSpec((NT, 1), lambda h: (0, 0)),
            pl.BlockSpec((1, N), lambda h: (0, 0)),
        ],
        out_specs=pl.BlockSpec((1, NT, DH), lambda h: (h, 0, 0)),
        out_shape=jax.ShapeDtypeStruct((NH, NT, DH), jnp.float32),
    )(q, k, v, widq, widk)

    attn2 = attn.transpose(1, 0, 2).reshape(NT, C)

    out = pl.pallas_call(
        _fuse_body,
        out_shape=jax.ShapeDtypeStruct((NT, C), jnp.float32),
    )(attn2, feats_t2, time_emb[2:3], Wproj, bproj.reshape(1, C),
      gamma.reshape(1, C), beta.reshape(1, C), W1, b1.reshape(1, 2 * C),
      W2, b2.reshape(1, C))

    return out, indices_t2
